# Initial kernel scaffold; baseline (speedup 1.0000x reference)
import jax
import jax.numpy as jnp
from jax.experimental import pallas as pl


def kernel(x, edge_index, W_l, W_r, b_l, bn_gamma, bn_beta):
    raise NotImplementedError("write your pallas kernel here")


# speedup vs baseline: 2.7717x; 2.7717x over previous
import functools

import jax
import jax.numpy as jnp
from jax import lax
from jax.experimental import pallas as pl
from jax.experimental.pallas import tpu as pltpu
from jax.experimental.pallas import tpu_sc as plsc

N = 10000
E = 320000
D = 128

NC = 2
NS = 16
NW = NC * NS
CHUNK = 128
ROWS_PER_TILE = 80
E_PAD = NW * ROWS_PER_TILE * CHUNK
N_ACC = 10240
ZROWS_SC = N_ACC // NS


def _sc_aggregate(x, src1, dst1):
    mesh = plsc.VectorSubcoreMesh(
        core_axis_name="c", subcore_axis_name="s", num_cores=NC,
        num_subcores=NS)

    @functools.partial(
        pl.kernel,
        out_type=[
            jax.ShapeDtypeStruct((NC, N_ACC, D), jnp.float32),
            jax.ShapeDtypeStruct((NC, N_ACC, D), jnp.float32),
        ],
        mesh=mesh,
        scratch_types=[
            pltpu.VMEM((CHUNK,), jnp.int32),
            pltpu.VMEM((CHUNK,), jnp.int32),
            pltpu.VMEM((CHUNK, D), jnp.float32),
            pltpu.VMEM((CHUNK, D), jnp.float32),
            pltpu.VMEM_SHARED((N_ACC, D), jnp.float32),
            pltpu.SemaphoreType.DMA,
        ],
    )
    def sc_kernel(x_hbm, src_hbm, dst_hbm, p_out, c_out,
                  sidx, didx, rows, ones, acc, sem):
        cid = lax.axis_index("c")
        sid = lax.axis_index("s")
        w = cid * NS + sid

        zeros16 = jnp.zeros((16,), jnp.float32)
        ones16 = jnp.ones((16,), jnp.float32)

        def fill_row(i, _):
            def fill_lane(j, _):
                rows[i, pl.ds(j * 16, 16)] = zeros16
                return 0
            lax.fori_loop(0, D // 16, fill_lane, 0)

            return 0
        lax.fori_loop(0, CHUNK, fill_row, 0)

        def fill_ones(i, _):
            def fill_lane(j, _):
                ones[i, pl.ds(j * 16, 16)] = ones16
                return 0
            lax.fori_loop(0, D // 16, fill_lane, 0)
            return 0
        lax.fori_loop(0, CHUNK, fill_ones, 0)

        zbase = sid * ZROWS_SC
        for t in range(ZROWS_SC // CHUNK):
            pltpu.sync_copy(rows, acc.at[pl.ds(zbase + t * CHUNK, CHUNK)])

        plsc.subcore_barrier()

        def body(k, _):
            ebase = (w * ROWS_PER_TILE + k) * CHUNK
            pltpu.sync_copy(src_hbm.at[pl.ds(ebase, CHUNK)], sidx)
            pltpu.sync_copy(dst_hbm.at[pl.ds(ebase, CHUNK)], didx)
            pltpu.async_copy(x_hbm.at[sidx], rows, sem).wait()
            pltpu.sync_copy(rows, acc.at[didx], add=True)
            return 0
        lax.fori_loop(0, ROWS_PER_TILE, body, 0)

        plsc.subcore_barrier()

        def fill_zero_rows(i, _):
            def fill_lane(j, _):
                rows[i, pl.ds(j * 16, 16)] = zeros16
                return 0
            lax.fori_loop(0, D // 16, fill_lane, 0)
            return 0

        for t in range(ZROWS_SC // CHUNK):
            off = zbase + t * CHUNK
            pltpu.sync_copy(acc.at[pl.ds(off, CHUNK)], rows)
            pltpu.sync_copy(rows, p_out.at[cid, pl.ds(off, CHUNK)])
        lax.fori_loop(0, CHUNK, fill_zero_rows, 0)
        for t in range(ZROWS_SC // CHUNK):
            pltpu.sync_copy(rows, acc.at[pl.ds(zbase + t * CHUNK, CHUNK)])

        plsc.subcore_barrier()

        def body2(k, _):
            ebase = (w * ROWS_PER_TILE + k) * CHUNK
            pltpu.sync_copy(dst_hbm.at[pl.ds(ebase, CHUNK)], didx)
            pltpu.sync_copy(ones, acc.at[didx], add=True)
            return 0
        lax.fori_loop(0, ROWS_PER_TILE, body2, 0)

        plsc.subcore_barrier()

        for t in range(ZROWS_SC // CHUNK):
            off = zbase + t * CHUNK
            pltpu.sync_copy(acc.at[pl.ds(off, CHUNK)], rows)
            pltpu.sync_copy(rows, c_out.at[cid, pl.ds(off, CHUNK)])

    return sc_kernel(x, src1, dst1)


_BN_INV = 1.0 / (1.0 + 1e-5) ** 0.5
_BLK = 400


def _tc_body(x_ref, p0_ref, p1_ref, c0_ref, c1_ref, wlt_ref, wrt_ref,
             b_ref, g_ref, bt_ref, o_ref):
    summed = p0_ref[...] + p1_ref[...]
    cnt = c0_ref[...] + c1_ref[...]
    mean = summed / jnp.maximum(cnt, 1.0)
    x = x_ref[...]
    h = (jnp.dot(mean, wlt_ref[...], preferred_element_type=jnp.float32)
         + jnp.dot(x, wrt_ref[...], preferred_element_type=jnp.float32)
         + b_ref[...])
    act = jnp.maximum(h, 0.0)
    o_ref[...] = x + act * (g_ref[...] * _BN_INV) + bt_ref[...]


def _tc_finish(x, p0, p1, c0, c1, wlt, wrt, b, g, bt):
    grid = (N // _BLK,)
    row_spec = pl.BlockSpec((_BLK, D), lambda i: (i, 0))
    full_spec = pl.BlockSpec((D, D), lambda i: (0, 0))
    vec_spec = pl.BlockSpec((1, D), lambda i: (0, 0))
    return pl.pallas_call(
        _tc_body,
        grid=grid,
        in_specs=[row_spec, row_spec, row_spec, row_spec, row_spec,
                  full_spec, full_spec, vec_spec, vec_spec, vec_spec],
        out_specs=row_spec,
        out_shape=jax.ShapeDtypeStruct((N, D), jnp.float32),
    )(x, p0, p1, c0, c1, wlt, wrt, b, g, bt)


def kernel(x, edge_index, W_l, W_r, b_l, bn_gamma, bn_beta):
    pad = E_PAD - E
    src = jnp.concatenate([edge_index[0], jnp.zeros((pad,), jnp.int32)])
    dst = jnp.concatenate([edge_index[1], jnp.full((pad,), N, jnp.int32)])

    p, c = _sc_aggregate(x, src, dst)

    out = _tc_finish(
        x,
        p[0, :N], p[1, :N], c[0, :N], c[1, :N],
        W_l.T, W_r.T,
        b_l.reshape(1, D), bn_gamma.reshape(1, D), bn_beta.reshape(1, D),
    )
    return out


# speedup vs baseline: 3.1595x; 1.1399x over previous
import functools

import jax
import jax.numpy as jnp
from jax import lax
from jax.experimental import pallas as pl
from jax.experimental.pallas import tpu as pltpu
from jax.experimental.pallas import tpu_sc as plsc

N = 10000
E = 320000
D = 128

NC = 2
NS = 16
NW = NC * NS
CHUNK = 128
ROWS_PER_TILE = 80
E_PAD = NW * ROWS_PER_TILE * CHUNK
N_ACC = 10240
ZROWS_SC = N_ACC // NS


def _sc_aggregate(x, src1, dst1):
    mesh = plsc.VectorSubcoreMesh(
        core_axis_name="c", subcore_axis_name="s", num_cores=NC,
        num_subcores=NS)

    @functools.partial(
        pl.kernel,
        out_type=[
            jax.ShapeDtypeStruct((NC, N_ACC, D), jnp.float32),
            jax.ShapeDtypeStruct((NC, N_ACC, D), jnp.float32),
        ],
        mesh=mesh,
        scratch_types=[
            pltpu.VMEM((2 * CHUNK,), jnp.int32),
            pltpu.VMEM((ROWS_PER_TILE, CHUNK), jnp.int32),
            pltpu.VMEM((CHUNK, D), jnp.float32),
            pltpu.VMEM((CHUNK, D), jnp.float32),
            pltpu.VMEM_SHARED((N_ACC, D), jnp.float32),
            pltpu.SemaphoreType.DMA,
            pltpu.SemaphoreType.DMA,
        ],
    )
    def sc_kernel(x_hbm, src_hbm, dst_hbm, p_out, c_out,
                  sidx, didx, r0, r1, acc, s0, s1):
        rbufs = (r0, r1)
        sems = (s0, s1)
        rows = r0
        ones = r1
        cid = lax.axis_index("c")
        sid = lax.axis_index("s")
        w = cid * NS + sid

        zeros16 = jnp.zeros((16,), jnp.float32)
        ones16 = jnp.ones((16,), jnp.float32)

        def fill_row(i, _):
            def fill_lane(j, _):
                rows[i, pl.ds(j * 16, 16)] = zeros16
                return 0
            lax.fori_loop(0, D // 16, fill_lane, 0)
            return 0
        lax.fori_loop(0, CHUNK, fill_row, 0)

        zbase = sid * ZROWS_SC
        for t in range(ZROWS_SC // CHUNK):
            pltpu.sync_copy(rows, acc.at[pl.ds(zbase + t * CHUNK, CHUNK)])

        pltpu.sync_copy(dst_hbm.at[w], didx)

        plsc.subcore_barrier()

        def body(g, _):
            pltpu.sync_copy(
                src_hbm.at[pl.ds((w * ROWS_PER_TILE + g * 2) * CHUNK,
                                 2 * CHUNK)], sidx)
            descs = []
            for b in range(2):
                descs.append(pltpu.async_copy(
                    x_hbm.at[sidx.at[pl.ds(b * CHUNK, CHUNK)]],
                    rbufs[b], sems[b]))
            for b in range(2):
                descs[b].wait()
                pltpu.sync_copy(rbufs[b], acc.at[didx.at[g * 2 + b]],
                                add=True)
            return 0
        lax.fori_loop(0, ROWS_PER_TILE // 2, body, 0)

        plsc.subcore_barrier()

        def fill_zero_rows(i, _):
            def fill_lane(j, _):
                rows[i, pl.ds(j * 16, 16)] = zeros16
                return 0
            lax.fori_loop(0, D // 16, fill_lane, 0)
            return 0

        for t in range(ZROWS_SC // CHUNK):
            off = zbase + t * CHUNK
            pltpu.sync_copy(acc.at[pl.ds(off, CHUNK)], rows)
            pltpu.sync_copy(rows, p_out.at[cid, pl.ds(off, CHUNK)])
        lax.fori_loop(0, CHUNK, fill_zero_rows, 0)
        for t in range(ZROWS_SC // CHUNK):
            pltpu.sync_copy(rows, acc.at[pl.ds(zbase + t * CHUNK, CHUNK)])

        def fill_ones(i, _):
            def fill_lane(j, _):
                ones[i, pl.ds(j * 16, 16)] = ones16
                return 0
            lax.fori_loop(0, D // 16, fill_lane, 0)
            return 0
        lax.fori_loop(0, CHUNK, fill_ones, 0)

        plsc.subcore_barrier()

        def body2(g, _):
            descs = [pltpu.async_copy(ones, acc.at[didx.at[g * 2 + b]],
                                      sems[b], add=True)
                     for b in range(2)]
            for d in descs:
                d.wait()
            return 0
        lax.fori_loop(0, ROWS_PER_TILE // 2, body2, 0)

        plsc.subcore_barrier()

        for t in range(ZROWS_SC // CHUNK):
            off = zbase + t * CHUNK
            pltpu.sync_copy(acc.at[pl.ds(off, CHUNK)], rows)
            pltpu.sync_copy(rows, c_out.at[cid, pl.ds(off, CHUNK)])

    return sc_kernel(x, src1, dst1)


_BN_INV = 1.0 / (1.0 + 1e-5) ** 0.5
_BLK = 400


def _tc_body(x_ref, p0_ref, p1_ref, c0_ref, c1_ref, wlt_ref, wrt_ref,
             b_ref, g_ref, bt_ref, o_ref):
    summed = p0_ref[...] + p1_ref[...]
    cnt = c0_ref[...] + c1_ref[...]
    mean = summed / jnp.maximum(cnt, 1.0)
    x = x_ref[...]
    h = (jnp.dot(mean, wlt_ref[...], preferred_element_type=jnp.float32)
         + jnp.dot(x, wrt_ref[...], preferred_element_type=jnp.float32)
         + b_ref[...])
    act = jnp.maximum(h, 0.0)
    o_ref[...] = x + act * (g_ref[...] * _BN_INV) + bt_ref[...]


def _tc_finish(x, p0, p1, c0, c1, wlt, wrt, b, g, bt):
    grid = (N // _BLK,)
    row_spec = pl.BlockSpec((_BLK, D), lambda i: (i, 0))
    full_spec = pl.BlockSpec((D, D), lambda i: (0, 0))
    vec_spec = pl.BlockSpec((1, D), lambda i: (0, 0))
    return pl.pallas_call(
        _tc_body,
        grid=grid,
        in_specs=[row_spec, row_spec, row_spec, row_spec, row_spec,
                  full_spec, full_spec, vec_spec, vec_spec, vec_spec],
        out_specs=row_spec,
        out_shape=jax.ShapeDtypeStruct((N, D), jnp.float32),
    )(x, p0, p1, c0, c1, wlt, wrt, b, g, bt)


def kernel(x, edge_index, W_l, W_r, b_l, bn_gamma, bn_beta):
    pad = E_PAD - E
    src = jnp.concatenate([edge_index[0], jnp.zeros((pad,), jnp.int32)])
    dst = jnp.concatenate(
        [edge_index[1], jnp.full((pad,), N, jnp.int32)]).reshape(
            NW, ROWS_PER_TILE, CHUNK)

    p, c = _sc_aggregate(x, src, dst)

    out = _tc_finish(
        x,
        p[0, :N], p[1, :N], c[0, :N], c[1, :N],
        W_l.T, W_r.T,
        b_l.reshape(1, D), bn_gamma.reshape(1, D), bn_beta.reshape(1, D),
    )
    return out


# speedup vs baseline: 3.4121x; 1.0799x over previous
import functools

import jax
import jax.numpy as jnp
from jax import lax
from jax.experimental import pallas as pl
from jax.experimental.pallas import tpu as pltpu
from jax.experimental.pallas import tpu_sc as plsc

N = 10000
E = 320000
D = 128

NC = 2
NS = 16
NW = NC * NS
CHUNK = 128
ROWS_PER_TILE = 80
E_PAD = NW * ROWS_PER_TILE * CHUNK
N_ACC = 10240
ZROWS_SC = N_ACC // NS


def _sc_aggregate(x, src1, dst1):
    mesh = plsc.VectorSubcoreMesh(
        core_axis_name="c", subcore_axis_name="s", num_cores=NC,
        num_subcores=NS)

    @functools.partial(
        pl.kernel,
        out_type=[
            jax.ShapeDtypeStruct((NC, N_ACC, D), jnp.float32),
            jax.ShapeDtypeStruct((NC, N_ACC, D), jnp.float32),
        ],
        mesh=mesh,
        scratch_types=[
            pltpu.VMEM((2 * CHUNK,), jnp.int32),
            pltpu.VMEM((2 * CHUNK,), jnp.int32),
            pltpu.VMEM((ROWS_PER_TILE, CHUNK), jnp.int32),
            pltpu.VMEM((CHUNK, D), jnp.float32),
            pltpu.VMEM((CHUNK, D), jnp.float32),
            pltpu.VMEM_SHARED((N_ACC, D), jnp.float32),
            pltpu.SemaphoreType.DMA,
            pltpu.SemaphoreType.DMA,
            pltpu.SemaphoreType.DMA,
            pltpu.SemaphoreType.DMA,
            pltpu.SemaphoreType.DMA,
            pltpu.SemaphoreType.DMA,
        ],
    )
    def sc_kernel(x_hbm, src_hbm, dst_hbm, p_out, c_out,
                  ia, ib, didx, r0, r1, acc, g0, g1, t0, t1, ja, jb):
        rbufs = (r0, r1)
        gsems = (g0, g1)
        tsems = (t0, t1)
        rows = r0
        ones = r1
        cid = lax.axis_index("c")
        sid = lax.axis_index("s")
        w = cid * NS + sid

        zeros16 = jnp.zeros((16,), jnp.float32)
        ones16 = jnp.ones((16,), jnp.float32)

        def fill_row(i, _):
            def fill_lane(j, _):
                rows[i, pl.ds(j * 16, 16)] = zeros16
                return 0
            lax.fori_loop(0, D // 16, fill_lane, 0)
            return 0
        lax.fori_loop(0, CHUNK, fill_row, 0)

        zbase = sid * ZROWS_SC
        for t in range(ZROWS_SC // CHUNK):
            pltpu.sync_copy(rows, acc.at[pl.ds(zbase + t * CHUNK, CHUNK)])

        pltpu.sync_copy(dst_hbm.at[w], didx)

        plsc.subcore_barrier()

        NPAIR = ROWS_PER_TILE // 2

        def idx_off(p):
            return jnp.minimum((w * ROWS_PER_TILE + p * 2) * CHUNK,
                               E_PAD - 2 * CHUNK)

        def start_idx(p, buf, sem):
            pltpu.async_copy(src_hbm.at[pl.ds(idx_off(p), 2 * CHUNK)],
                             buf, sem)

        def wait_idx(buf, sem):
            pltpu.make_async_copy(src_hbm.at[pl.ds(0, 2 * CHUNK)],
                                  buf, sem).wait()

        def start_gather(b, sbuf, half):
            pltpu.async_copy(x_hbm.at[sbuf.at[pl.ds(half * CHUNK, CHUNK)]],
                             rbufs[b], gsems[b])

        def wait_gather(b):
            pltpu.make_async_copy(x_hbm.at[pl.ds(0, CHUNK)],
                                  rbufs[b], gsems[b]).wait()

        def start_scatter(b, k):
            pltpu.async_copy(rbufs[b], acc.at[didx.at[k]], tsems[b],
                             add=True)

        def wait_scatter(b):
            pltpu.make_async_copy(x_hbm.at[pl.ds(0, CHUNK)],
                                  rbufs[b], tsems[b]).wait()

        def emit_pair(p, cur, nxt, nxt_sem, cur_sem, issue_next):
            wait_gather(0)
            start_scatter(0, 2 * p)
            wait_gather(1)
            start_scatter(1, 2 * p + 1)
            if issue_next:
                wait_idx(nxt, nxt_sem)
                wait_scatter(0)
                start_gather(0, nxt, 0)
                wait_scatter(1)
                start_gather(1, nxt, 1)
                start_idx(p + 2, cur, cur_sem)
            else:
                wait_scatter(0)
                wait_scatter(1)

        pltpu.sync_copy(src_hbm.at[pl.ds(w * ROWS_PER_TILE * CHUNK,
                                         2 * CHUNK)], ia)
        start_gather(0, ia, 0)
        start_gather(1, ia, 1)
        start_idx(1, ib, jb)

        def body(gg, _):
            emit_pair(2 * gg, ia, ib, jb, ja, True)
            emit_pair(2 * gg + 1, ib, ia, ja, jb, True)
            return 0
        lax.fori_loop(0, NPAIR // 2 - 1, body, 0)

        emit_pair(NPAIR - 2, ia, ib, jb, ja, True)
        emit_pair(NPAIR - 1, ib, ia, ja, jb, False)
        wait_idx(ia, ja)

        plsc.subcore_barrier()

        def fill_zero_rows(i, _):
            def fill_lane(j, _):
                rows[i, pl.ds(j * 16, 16)] = zeros16
                return 0
            lax.fori_loop(0, D // 16, fill_lane, 0)
            return 0

        for t in range(ZROWS_SC // CHUNK):
            off = zbase + t * CHUNK
            pltpu.sync_copy(acc.at[pl.ds(off, CHUNK)], rows)
            pltpu.sync_copy(rows, p_out.at[cid, pl.ds(off, CHUNK)])
        lax.fori_loop(0, CHUNK, fill_zero_rows, 0)
        for t in range(ZROWS_SC // CHUNK):
            pltpu.sync_copy(rows, acc.at[pl.ds(zbase + t * CHUNK, CHUNK)])

        def fill_ones(i, _):
            def fill_lane(j, _):
                ones[i, pl.ds(j * 16, 16)] = ones16
                return 0
            lax.fori_loop(0, D // 16, fill_lane, 0)
            return 0
        lax.fori_loop(0, CHUNK, fill_ones, 0)

        plsc.subcore_barrier()

        def start_cnt(b, k):
            pltpu.async_copy(ones, acc.at[didx.at[k]], tsems[b], add=True)

        def wait_cnt(b):
            pltpu.make_async_copy(x_hbm.at[pl.ds(0, CHUNK)],
                                  ones, tsems[b]).wait()

        start_cnt(0, 0)
        start_cnt(1, 1)

        def body2(k2, _):
            wait_cnt(0)
            start_cnt(0, 2 * k2 + 2)
            wait_cnt(1)
            start_cnt(1, 2 * k2 + 3)
            return 0
        lax.fori_loop(0, ROWS_PER_TILE // 2 - 1, body2, 0)
        wait_cnt(0)
        wait_cnt(1)

        plsc.subcore_barrier()

        for t in range(ZROWS_SC // CHUNK):
            off = zbase + t * CHUNK
            pltpu.sync_copy(acc.at[pl.ds(off, CHUNK)], rows)
            pltpu.sync_copy(rows, c_out.at[cid, pl.ds(off, CHUNK)])

    return sc_kernel(x, src1, dst1)


_BN_INV = 1.0 / (1.0 + 1e-5) ** 0.5
_BLK = 400


def _tc_body(x_ref, p0_ref, p1_ref, c0_ref, c1_ref, wlt_ref, wrt_ref,
             b_ref, g_ref, bt_ref, o_ref):
    summed = p0_ref[...] + p1_ref[...]
    cnt = c0_ref[...] + c1_ref[...]
    mean = summed / jnp.maximum(cnt, 1.0)
    x = x_ref[...]
    h = (jnp.dot(mean, wlt_ref[...], preferred_element_type=jnp.float32)
         + jnp.dot(x, wrt_ref[...], preferred_element_type=jnp.float32)
         + b_ref[...])
    act = jnp.maximum(h, 0.0)
    o_ref[...] = x + act * (g_ref[...] * _BN_INV) + bt_ref[...]


def _tc_finish(x, p0, p1, c0, c1, wlt, wrt, b, g, bt):
    grid = (N // _BLK,)
    row_spec = pl.BlockSpec((_BLK, D), lambda i: (i, 0))
    full_spec = pl.BlockSpec((D, D), lambda i: (0, 0))
    vec_spec = pl.BlockSpec((1, D), lambda i: (0, 0))
    return pl.pallas_call(
        _tc_body,
        grid=grid,
        in_specs=[row_spec, row_spec, row_spec, row_spec, row_spec,
                  full_spec, full_spec, vec_spec, vec_spec, vec_spec],
        out_specs=row_spec,
        out_shape=jax.ShapeDtypeStruct((N, D), jnp.float32),
    )(x, p0, p1, c0, c1, wlt, wrt, b, g, bt)


def kernel(x, edge_index, W_l, W_r, b_l, bn_gamma, bn_beta):
    pad = E_PAD - E
    src = jnp.concatenate([edge_index[0], jnp.zeros((pad,), jnp.int32)])
    dst = jnp.concatenate(
        [edge_index[1], jnp.full((pad,), N, jnp.int32)]).reshape(
            NW, ROWS_PER_TILE, CHUNK)

    p, c = _sc_aggregate(x, src, dst)

    out = _tc_finish(
        x,
        p[0, :N], p[1, :N], c[0, :N], c[1, :N],
        W_l.T, W_r.T,
        b_l.reshape(1, D), bn_gamma.reshape(1, D), bn_beta.reshape(1, D),
    )
    return out


# speedup vs baseline: 3.4164x; 1.0012x over previous
import functools

import jax
import jax.numpy as jnp
from jax import lax
from jax.experimental import pallas as pl
from jax.experimental.pallas import tpu as pltpu
from jax.experimental.pallas import tpu_sc as plsc

N = 10000
E = 320000
D = 128

NC = 2
NS = 16
NW = NC * NS
CHUNK = 128
ROWS_PER_TILE = 80
E_PAD = NW * ROWS_PER_TILE * CHUNK
N_ACC = 10240
ZROWS_SC = N_ACC // NS


def _sc_aggregate(x, src1, dst1):
    mesh = plsc.VectorSubcoreMesh(
        core_axis_name="c", subcore_axis_name="s", num_cores=NC,
        num_subcores=NS)

    @functools.partial(
        pl.kernel,
        out_type=[
            jax.ShapeDtypeStruct((NC, N_ACC, D), jnp.float32),
            jax.ShapeDtypeStruct((NC, N_ACC, D), jnp.float32),
        ],
        mesh=mesh,
        scratch_types=[
            pltpu.VMEM((2 * CHUNK,), jnp.int32),
            pltpu.VMEM((2 * CHUNK,), jnp.int32),
            pltpu.VMEM((ROWS_PER_TILE, CHUNK), jnp.int32),
            pltpu.VMEM((CHUNK, D), jnp.float32),
            pltpu.VMEM((CHUNK, D), jnp.float32),
            pltpu.VMEM_SHARED((N_ACC, D), jnp.float32),
            pltpu.SemaphoreType.DMA,
            pltpu.SemaphoreType.DMA,
            pltpu.SemaphoreType.DMA,
            pltpu.SemaphoreType.DMA,
            pltpu.SemaphoreType.DMA,
            pltpu.SemaphoreType.DMA,
        ],
    )
    def sc_kernel(x_hbm, src_hbm, dst_hbm, p_out, c_out,
                  ia, ib, didx, r0, r1, acc, g0, g1, t0, t1, ja, jb):
        rbufs = (r0, r1)
        gsems = (g0, g1)
        tsems = (t0, t1)
        rows = r0
        ones = r1
        cid = lax.axis_index("c")
        sid = lax.axis_index("s")
        w = cid * NS + sid

        zeros16 = jnp.zeros((16,), jnp.float32)
        ones16 = jnp.ones((16,), jnp.float32)

        def fill_row(i, _):
            def fill_lane(j, _):
                rows[i, pl.ds(j * 16, 16)] = zeros16
                return 0
            lax.fori_loop(0, D // 16, fill_lane, 0)
            return 0
        lax.fori_loop(0, CHUNK, fill_row, 0)

        zbase = sid * ZROWS_SC
        for t in range(ZROWS_SC // CHUNK):
            pltpu.sync_copy(rows, acc.at[pl.ds(zbase + t * CHUNK, CHUNK)])

        pltpu.sync_copy(dst_hbm.at[w], didx)

        plsc.subcore_barrier()

        NPAIR = ROWS_PER_TILE // 2

        def idx_off(p):
            return jnp.minimum((w * ROWS_PER_TILE + p * 2) * CHUNK,
                               E_PAD - 2 * CHUNK)

        def start_idx(p, buf, sem):
            pltpu.async_copy(src_hbm.at[pl.ds(idx_off(p), 2 * CHUNK)],
                             buf, sem)

        def wait_idx(buf, sem):
            pltpu.make_async_copy(src_hbm.at[pl.ds(0, 2 * CHUNK)],
                                  buf, sem).wait()

        def start_gather(b, sbuf, half):
            for q in range(4):
                pltpu.async_copy(
                    x_hbm.at[sbuf.at[pl.ds(half * CHUNK + q * 32, 32)]],
                    rbufs[b].at[pl.ds(q * 32, 32)], gsems[b])

        def wait_gather(b):
            pltpu.make_async_copy(x_hbm.at[pl.ds(0, CHUNK)],
                                  rbufs[b], gsems[b]).wait()

        def start_scatter(b, k):
            pltpu.async_copy(rbufs[b], acc.at[didx.at[k]], tsems[b],
                             add=True)

        def wait_scatter(b):
            pltpu.make_async_copy(x_hbm.at[pl.ds(0, CHUNK)],
                                  rbufs[b], tsems[b]).wait()

        def emit_pair(p, cur, nxt, nxt_sem, cur_sem, issue_next):
            wait_gather(0)
            start_scatter(0, 2 * p)
            wait_gather(1)
            start_scatter(1, 2 * p + 1)
            if issue_next:
                wait_idx(nxt, nxt_sem)
                wait_scatter(0)
                start_gather(0, nxt, 0)
                wait_scatter(1)
                start_gather(1, nxt, 1)
                start_idx(p + 2, cur, cur_sem)
            else:
                wait_scatter(0)
                wait_scatter(1)

        pltpu.sync_copy(src_hbm.at[pl.ds(w * ROWS_PER_TILE * CHUNK,
                                         2 * CHUNK)], ia)
        start_gather(0, ia, 0)
        start_gather(1, ia, 1)
        start_idx(1, ib, jb)

        def body(gg, _):
            emit_pair(2 * gg, ia, ib, jb, ja, True)
            emit_pair(2 * gg + 1, ib, ia, ja, jb, True)
            return 0
        lax.fori_loop(0, NPAIR // 2 - 1, body, 0)

        emit_pair(NPAIR - 2, ia, ib, jb, ja, True)
        emit_pair(NPAIR - 1, ib, ia, ja, jb, False)
        wait_idx(ia, ja)

        plsc.subcore_barrier()

        def fill_zero_rows(i, _):
            def fill_lane(j, _):
                rows[i, pl.ds(j * 16, 16)] = zeros16
                return 0
            lax.fori_loop(0, D // 16, fill_lane, 0)
            return 0

        for t in range(ZROWS_SC // CHUNK):
            off = zbase + t * CHUNK
            pltpu.sync_copy(acc.at[pl.ds(off, CHUNK)], rows)
            pltpu.sync_copy(rows, p_out.at[cid, pl.ds(off, CHUNK)])
        lax.fori_loop(0, CHUNK, fill_zero_rows, 0)
        for t in range(ZROWS_SC // CHUNK):
            pltpu.sync_copy(rows, acc.at[pl.ds(zbase + t * CHUNK, CHUNK)])

        def fill_ones(i, _):
            def fill_lane(j, _):
                ones[i, pl.ds(j * 16, 16)] = ones16
                return 0
            lax.fori_loop(0, D // 16, fill_lane, 0)
            return 0
        lax.fori_loop(0, CHUNK, fill_ones, 0)

        plsc.subcore_barrier()

        csems = (t0, t1, ja, jb)

        def start_cnt(b, k):
            pltpu.async_copy(ones, acc.at[didx.at[k]], csems[b], add=True)

        def wait_cnt(b):
            pltpu.make_async_copy(x_hbm.at[pl.ds(0, CHUNK)],
                                  ones, csems[b]).wait()

        for b in range(4):
            start_cnt(b, b)

        def body2(k4, _):
            for b in range(4):
                wait_cnt(b)
                start_cnt(b, 4 * k4 + 4 + b)
            return 0
        lax.fori_loop(0, ROWS_PER_TILE // 4 - 1, body2, 0)
        for b in range(4):
            wait_cnt(b)

        plsc.subcore_barrier()

        for t in range(ZROWS_SC // CHUNK):
            off = zbase + t * CHUNK
            pltpu.sync_copy(acc.at[pl.ds(off, CHUNK)], rows)
            pltpu.sync_copy(rows, c_out.at[cid, pl.ds(off, CHUNK)])

    return sc_kernel(x, src1, dst1)


_BN_INV = 1.0 / (1.0 + 1e-5) ** 0.5
_BLK = 400


def _tc_body(x_ref, p0_ref, p1_ref, c0_ref, c1_ref, wlt_ref, wrt_ref,
             b_ref, g_ref, bt_ref, o_ref):
    summed = p0_ref[...] + p1_ref[...]
    cnt = c0_ref[...] + c1_ref[...]
    mean = summed / jnp.maximum(cnt, 1.0)
    x = x_ref[...]
    h = (jnp.dot(mean, wlt_ref[...], preferred_element_type=jnp.float32)
         + jnp.dot(x, wrt_ref[...], preferred_element_type=jnp.float32)
         + b_ref[...])
    act = jnp.maximum(h, 0.0)
    o_ref[...] = x + act * (g_ref[...] * _BN_INV) + bt_ref[...]


def _tc_finish(x, p0, p1, c0, c1, wlt, wrt, b, g, bt):
    grid = (N // _BLK,)
    row_spec = pl.BlockSpec((_BLK, D), lambda i: (i, 0))
    full_spec = pl.BlockSpec((D, D), lambda i: (0, 0))
    vec_spec = pl.BlockSpec((1, D), lambda i: (0, 0))
    return pl.pallas_call(
        _tc_body,
        grid=grid,
        in_specs=[row_spec, row_spec, row_spec, row_spec, row_spec,
                  full_spec, full_spec, vec_spec, vec_spec, vec_spec],
        out_specs=row_spec,
        out_shape=jax.ShapeDtypeStruct((N, D), jnp.float32),
    )(x, p0, p1, c0, c1, wlt, wrt, b, g, bt)


def kernel(x, edge_index, W_l, W_r, b_l, bn_gamma, bn_beta):
    pad = E_PAD - E
    src = jnp.concatenate([edge_index[0], jnp.zeros((pad,), jnp.int32)])
    dst = jnp.concatenate(
        [edge_index[1], jnp.full((pad,), N, jnp.int32)]).reshape(
            NW, ROWS_PER_TILE, CHUNK)

    p, c = _sc_aggregate(x, src, dst)

    out = _tc_finish(
        x,
        p[0, :N], p[1, :N], c[0, :N], c[1, :N],
        W_l.T, W_r.T,
        b_l.reshape(1, D), bn_gamma.reshape(1, D), bn_beta.reshape(1, D),
    )
    return out
